# baseline (speedup 1.0000x reference)
import functools

import jax
import jax.numpy as jnp
import numpy as np
from jax import lax
from jax.experimental import pallas as pl
from jax.experimental.pallas import tpu as pltpu
from jax.experimental.pallas import tpu_sc as plsc

N = 512
B = 128
T = N * (N + 1) // 2

_ROW_SPLIT = 144
_SIZE_A = _ROW_SPLIT * (2 * N + 1 - _ROW_SPLIT) // 2
_SIZE_B = T - _SIZE_A
_PIECES = (
    (0, _ROW_SPLIT // 16, 0, _SIZE_A),
    (_ROW_SPLIT, (N - _ROW_SPLIT) // 16, _SIZE_A, _SIZE_B),
)

_INFO = plsc.get_sparse_core_info()
_NW = _INFO.num_cores * _INFO.num_subcores
_BPW = B // _NW

_IN_BUF = 16 * N + 128
_PIECE_BUF = _SIZE_B + 128


def _sc_flatten_body(x_hbm, y_hbm, in_buf, piece_buf):
    wid = lax.axis_index("s") * _INFO.num_cores + lax.axis_index("c")

    def batch_body(k, carry0):
        b = wid * _BPW + k
        x_base = b * (N * N)
        y_base = b * T
        for row0, nchunks, pbase, psize in _PIECES:
            def chunk_body(cc, carry, row0=row0, pbase=pbase, x_base=x_base):
                i0 = row0 + cc * 16
                pltpu.sync_copy(
                    x_hbm.at[pl.ds(x_base + i0 * N, 16 * N)],
                    in_buf.at[pl.ds(0, 16 * N)],
                )
                for r in range(16):
                    i = i0 + r
                    dstloc = N * i - (i * (i - 1)) // 2 - pbase
                    srcloc = N * r + i
                    nch = (N + 127 - i) >> 7

                    def copy_body(c, carry2, srcloc=srcloc, dstloc=dstloc):
                        bs = srcloc + c * 128
                        bd = dstloc + c * 128
                        for q in range(8):
                            piece_buf[pl.ds(bd + q * 16, 16)] = (
                                in_buf[pl.ds(bs + q * 16, 16)]
                            )
                        return carry2

                    lax.fori_loop(0, nch, copy_body, 0)
                return carry

            lax.fori_loop(0, nchunks, chunk_body, 0)
            pltpu.sync_copy(
                piece_buf.at[pl.ds(0, psize)],
                y_hbm.at[pl.ds(y_base + pbase, psize)],
            )
        return carry0

    lax.fori_loop(0, _BPW, batch_body, 0)


_sc_flatten = functools.partial(
    pl.kernel,
    mesh=plsc.VectorSubcoreMesh(core_axis_name="c", subcore_axis_name="s"),
    out_type=jax.ShapeDtypeStruct((B * T,), jnp.float32),
    scratch_types=[
        pltpu.VMEM((_IN_BUF,), jnp.float32),
        pltpu.VMEM((_PIECE_BUF,), jnp.float32),
    ],
)(_sc_flatten_body)


_ROW_NP, _COL_NP = np.triu_indices(N, k=0)
_SCALE_NP = np.where(
    _ROW_NP != _COL_NP, np.sqrt(np.float32(2.0)), np.float32(1.0)
).astype(np.float32)[None, :]

_TC_ROWS = 8


def _tc_norm_body(scale_ref, y_ref, o_ref):
    w = y_ref[...]
    t = jnp.abs(w) * scale_ref[...]
    y0 = jnp.sign(w) * jnp.sqrt(t)
    nrm = jnp.sqrt(jnp.sum(y0 * y0, axis=1, keepdims=True))
    o_ref[...] = y0 / nrm


_tc_norm = pl.pallas_call(
    _tc_norm_body,
    grid=(B // _TC_ROWS,),
    in_specs=[
        pl.BlockSpec((1, T), lambda g: (0, 0)),
        pl.BlockSpec((_TC_ROWS, T), lambda g: (g, 0)),
    ],
    out_specs=pl.BlockSpec((_TC_ROWS, T), lambda g: (g, 0)),
    out_shape=jax.ShapeDtypeStruct((B, T), jnp.float32),
)


def kernel(x):
    y = _sc_flatten(x.reshape(-1))
    return _tc_norm(jnp.asarray(_SCALE_NP), y.reshape(B, T))


# speedup vs baseline: 1.3066x; 1.3066x over previous
import functools

import jax
import jax.numpy as jnp
import numpy as np
from jax import lax
from jax.experimental import pallas as pl
from jax.experimental.pallas import tpu as pltpu
from jax.experimental.pallas import tpu_sc as plsc

N = 512
B = 128
T = N * (N + 1) // 2

_SIZE_A = 144 * (2 * N + 1 - 144) // 2
_SIZE_B = T - _SIZE_A
_PIECES = (
    (0, _SIZE_A, ((0, 48), (48, 48), (96, 48))),
    (_SIZE_A, _SIZE_B, ((144, 48), (192, 48), (240, 48), (288, 48),
                        (336, 48), (384, 48), (432, 48), (480, 32))),
)
_CHUNKS = [c for _, _, cs in _PIECES for c in cs]

_INFO = plsc.get_sparse_core_info()
_NW = _INFO.num_cores * _INFO.num_subcores
_BPW = B // _NW

_IN_BUF = 48 * N + 128
_PIECE_BUF = _SIZE_B + 128


def _sc_flatten_body(x_hbm, y_hbm, in_a, in_b, piece_buf, sem_a, sem_b):
    wid = lax.axis_index("s") * _INFO.num_cores + lax.axis_index("c")
    bufs = (in_a, in_b)
    sems = (sem_a, sem_b)

    def load(ci, x_base):
        row0, nrows = _CHUNKS[ci]
        pltpu.make_async_copy(
            x_hbm.at[pl.ds(x_base + row0 * N, nrows * N)],
            bufs[ci % 2].at[pl.ds(0, nrows * N)],
            sems[ci % 2],
        ).start()

    def wait(ci):
        row0, nrows = _CHUNKS[ci]
        pltpu.make_async_copy(
            x_hbm.at[pl.ds(0, nrows * N)],
            bufs[ci % 2].at[pl.ds(0, nrows * N)],
            sems[ci % 2],
        ).wait()

    def batch_body(k, carry0):
        b = wid * _BPW + k
        x_base = b * (N * N)
        y_base = b * T
        load(0, x_base)
        ci = 0
        for pbase, psize, chunks in _PIECES:
            for row0, nrows in chunks:
                wait(ci)
                if ci + 1 < len(_CHUNKS):
                    load(ci + 1, x_base)
                buf = bufs[ci % 2]

                def row_body(r, carry1, row0=row0, buf=buf, pbase=pbase):
                    i = row0 + r
                    srcloc = N * r + i
                    dstloc = N * i - (i * (i - 1)) // 2 - pbase
                    nch = (N + 127 - i) >> 7

                    def copy_body(c, carry2):
                        bs = srcloc + c * 128
                        bd = dstloc + c * 128
                        for q in range(8):
                            piece_buf[pl.ds(bd + q * 16, 16)] = (
                                buf[pl.ds(bs + q * 16, 16)]
                            )
                        return carry2

                    return lax.fori_loop(0, nch, copy_body, carry1)

                lax.fori_loop(0, nrows, row_body, 0)
                ci += 1
            pltpu.sync_copy(
                piece_buf.at[pl.ds(0, psize)],
                y_hbm.at[pl.ds(y_base + pbase, psize)],
            )
        return carry0

    lax.fori_loop(0, _BPW, batch_body, 0)


_sc_flatten = functools.partial(
    pl.kernel,
    mesh=plsc.VectorSubcoreMesh(core_axis_name="c", subcore_axis_name="s"),
    out_type=jax.ShapeDtypeStruct((B * T,), jnp.float32),
    scratch_types=[
        pltpu.VMEM((_IN_BUF,), jnp.float32),
        pltpu.VMEM((_IN_BUF,), jnp.float32),
        pltpu.VMEM((_PIECE_BUF,), jnp.float32),
        pltpu.SemaphoreType.DMA,
        pltpu.SemaphoreType.DMA,
    ],
)(_sc_flatten_body)


_ROW_NP, _COL_NP = np.triu_indices(N, k=0)
_SCALE_NP = np.where(
    _ROW_NP != _COL_NP, np.sqrt(np.float32(2.0)), np.float32(1.0)
).astype(np.float32)[None, :]

_TC_ROWS = 8


def _tc_norm_body(scale_ref, y_ref, o_ref):
    w = y_ref[...]
    t = jnp.abs(w) * scale_ref[...]
    y0 = jnp.sign(w) * jnp.sqrt(t)
    nrm = jnp.sqrt(jnp.sum(y0 * y0, axis=1, keepdims=True))
    o_ref[...] = y0 / nrm


_tc_norm = pl.pallas_call(
    _tc_norm_body,
    grid=(B // _TC_ROWS,),
    in_specs=[
        pl.BlockSpec((1, T), lambda g: (0, 0)),
        pl.BlockSpec((_TC_ROWS, T), lambda g: (g, 0)),
    ],
    out_specs=pl.BlockSpec((_TC_ROWS, T), lambda g: (g, 0)),
    out_shape=jax.ShapeDtypeStruct((B, T), jnp.float32),
)


def kernel(x):
    y = _sc_flatten(x.reshape(-1))
    return _tc_norm(jnp.asarray(_SCALE_NP), y.reshape(B, T))


# speedup vs baseline: 1.3152x; 1.0066x over previous
import functools

import jax
import jax.numpy as jnp
import numpy as np
from jax import lax
from jax.experimental import pallas as pl
from jax.experimental.pallas import tpu as pltpu
from jax.experimental.pallas import tpu_sc as plsc

N = 512
B = 128
T = N * (N + 1) // 2

_SIZE_A = 144 * (2 * N + 1 - 144) // 2
_SIZE_B = T - _SIZE_A
_PIECES = (
    (0, _SIZE_A, ((0, 48), (48, 48), (96, 48))),
    (_SIZE_A, _SIZE_B, ((144, 48), (192, 48), (240, 48), (288, 48),
                        (336, 48), (384, 48), (432, 48), (480, 32))),
)
_CHUNKS = [c for _, _, cs in _PIECES for c in cs]

_INFO = plsc.get_sparse_core_info()
_NW = _INFO.num_cores * _INFO.num_subcores
_BPW = B // _NW

_IN_BUF = 48 * N + 128
_PIECE_BUF = _SIZE_B + 128


def _sc_flatten_body(x_hbm, y_hbm, in_a, in_b, piece_buf, sem_a, sem_b):
    wid = lax.axis_index("s") * _INFO.num_cores + lax.axis_index("c")
    bufs = (in_a, in_b)
    sems = (sem_a, sem_b)

    def load(ci, x_base):
        row0, nrows = _CHUNKS[ci]
        pltpu.make_async_copy(
            x_hbm.at[pl.ds(x_base + row0 * N, nrows * N)],
            bufs[ci % 2].at[pl.ds(0, nrows * N)],
            sems[ci % 2],
        ).start()

    def wait(ci):
        row0, nrows = _CHUNKS[ci]
        pltpu.make_async_copy(
            x_hbm.at[pl.ds(0, nrows * N)],
            bufs[ci % 2].at[pl.ds(0, nrows * N)],
            sems[ci % 2],
        ).wait()

    def batch_body(k, carry0):
        b = wid * _BPW + k
        x_base = b * (N * N)
        y_base = b * T
        load(0, x_base)
        ci = 0
        for pbase, psize, chunks in _PIECES:
            for row0, nrows in chunks:
                wait(ci)
                if ci + 1 < len(_CHUNKS):
                    load(ci + 1, x_base)
                buf = bufs[ci % 2]

                def row_body(r, carry1, row0=row0, buf=buf, pbase=pbase):
                    i = row0 + r
                    srcloc = N * r + i
                    dstloc = N * i - (i * (i - 1)) // 2 - pbase
                    nch = (N + 127 - i) >> 7

                    def copy_body(c, carry2):
                        bs = srcloc + c * 128
                        bd = dstloc + c * 128
                        for q in range(8):
                            piece_buf[pl.ds(bd + q * 16, 16)] = (
                                buf[pl.ds(bs + q * 16, 16)]
                            )
                        return carry2

                    return lax.fori_loop(0, nch, copy_body, carry1)

                lax.fori_loop(0, nrows, row_body, 0)
                ci += 1
            pltpu.sync_copy(
                piece_buf.at[pl.ds(0, psize)],
                y_hbm.at[pl.ds(y_base + pbase, psize)],
            )
        return carry0

    lax.fori_loop(0, _BPW, batch_body, 0)


_sc_flatten = functools.partial(
    pl.kernel,
    mesh=plsc.VectorSubcoreMesh(core_axis_name="c", subcore_axis_name="s"),
    out_type=jax.ShapeDtypeStruct((B * T,), jnp.float32),
    scratch_types=[
        pltpu.VMEM((_IN_BUF,), jnp.float32),
        pltpu.VMEM((_IN_BUF,), jnp.float32),
        pltpu.VMEM((_PIECE_BUF,), jnp.float32),
        pltpu.SemaphoreType.DMA,
        pltpu.SemaphoreType.DMA,
    ],
)(_sc_flatten_body)


_ROW_NP, _COL_NP = np.triu_indices(N, k=0)
_SCALE_NP = np.where(
    _ROW_NP != _COL_NP, np.sqrt(np.float32(2.0)), np.float32(1.0)
).astype(np.float32)[None, :]

_TC_ROWS = 16


def _tc_norm_body(scale_ref, y_ref, o_ref):
    w = y_ref[...]
    t = jnp.abs(w) * scale_ref[...]
    y0 = jnp.sign(w) * jnp.sqrt(t)
    nrm = jnp.sqrt(jnp.sum(y0 * y0, axis=1, keepdims=True))
    o_ref[...] = y0 / nrm


_tc_norm = pl.pallas_call(
    _tc_norm_body,
    grid=(B // _TC_ROWS,),
    in_specs=[
        pl.BlockSpec((1, T), lambda g: (0, 0)),
        pl.BlockSpec((_TC_ROWS, T), lambda g: (g, 0)),
    ],
    out_specs=pl.BlockSpec((_TC_ROWS, T), lambda g: (g, 0)),
    out_shape=jax.ShapeDtypeStruct((B, T), jnp.float32),
)


def kernel(x):
    y = _sc_flatten(x.reshape(-1))
    return _tc_norm(jnp.asarray(_SCALE_NP), y.reshape(B, T))
